# baseline (speedup 1.0000x reference)
import functools

import jax
import jax.numpy as jnp
from jax import lax
from jax.experimental import pallas as pl
from jax.experimental.pallas import tpu as pltpu
from jax.experimental.pallas import tpu_sc as plsc

_ANCHOR_W = (116.0, 156.0, 373.0)
_ANCHOR_H = (90.0, 198.0, 326.0)
_CIN = 89
_COUT = 85
_G = 52
_RCHUNK = 8
_HP = 4 * _G
_FULL_CHUNKS = _G // _RCHUNK
_TAIL_R0 = _FULL_CHUNKS * _RCHUNK
_NA = 56
_NB = _CIN - _NA


def _rcp(d):
    y = plsc.bitcast(jnp.int32(0x7EF311C3) - plsc.bitcast(d, jnp.int32),
                     jnp.float32)
    y = y * (2.0 - d * y)
    return y * (2.0 - d * y)


def _sigmoid(v):
    return _rcp(1.0 + jnp.exp(-v))


def _sc_body(nslab, x_hbm, st_hbm, out_hbm, bufa, bufb, obuf, stv, sem):
    nc = 2
    wid = lax.axis_index("s") * nc + lax.axis_index("c")
    pltpu.sync_copy(st_hbm, stv)
    i16 = lax.iota(jnp.int32, 16)

    def process_chunk(slab, r0, nrows):
        b = slab // 3
        a = slab - b * 3
        aw = jnp.where(a == 0, _ANCHOR_W[0],
                       jnp.where(a == 1, _ANCHOR_W[1], _ANCHOR_W[2]))
        ah = jnp.where(a == 0, _ANCHOR_H[0],
                       jnp.where(a == 1, _ANCHOR_H[1], _ANCHOR_H[2]))
        st = stv[:]

        dst_a = bufa if nrows == _RCHUNK else bufa.at[:, pl.ds(0, nrows)]
        dst_b = bufb if nrows == _RCHUNK else bufb.at[:, pl.ds(0, nrows)]
        ha = pltpu.async_copy(
            x_hbm.at[b, pl.ds(a * _CIN, _NA), pl.ds(r0, nrows), :], dst_a, sem)
        hb = pltpu.async_copy(
            x_hbm.at[b, pl.ds(a * _CIN + _NA, _NB), pl.ds(r0, nrows), :],
            dst_b, sem)
        ha.wait()
        hb.wait()

        for h in range(nrows // 4):
            p0 = h * _HP

            def j_body(j, carry2):
                p = i16 + j * 16
                pg = p + p0
                rv = pg // _G
                lv = pg - rv * _G
                z = jnp.full(16, 0, jnp.int32)

                s0 = _sigmoid(plsc.load_gather(bufa, [z, rv, lv]))
                s1 = _sigmoid(plsc.load_gather(bufa, [z + 1, rv, lv]))
                e2 = jnp.exp(plsc.load_gather(bufa, [z + 2, rv, lv]))
                e3 = jnp.exp(plsc.load_gather(bufa, [z + 3, rv, lv]))
                c84 = _sigmoid(plsc.load_gather(bufb, [z + 32, rv, lv]))
                plsc.store_scatter(obuf, [p, z],
                                   (s0 + lv.astype(jnp.float32)) * st)
                plsc.store_scatter(obuf, [p, z + 1],
                                   (s1 + (rv + r0).astype(jnp.float32)) * st)
                plsc.store_scatter(obuf, [p, z + 2], e2 * aw)
                plsc.store_scatter(obuf, [p, z + 3], e3 * ah)
                plsc.store_scatter(obuf, [p, z + 84], c84)

                @plsc.parallel_loop(0, _NA - 8, 1, unroll=4)
                def ca_loop(c):
                    v = _sigmoid(plsc.load_gather(bufa, [z + (c + 8), rv, lv]))
                    plsc.store_scatter(obuf, [p, z + (c + 4)], v)

                @plsc.parallel_loop(0, _NB - 1, 1, unroll=4)
                def cb_loop(c):
                    v = _sigmoid(plsc.load_gather(bufb, [z + c, rv, lv]))
                    plsc.store_scatter(obuf, [p, z + (c + 52)], v)

                return carry2

            lax.fori_loop(0, _HP // 16, j_body, 0)

            n0 = a * (_G * _G) + r0 * _G + p0
            pltpu.sync_copy(obuf, out_hbm.at[b, pl.ds(n0, _HP), :])

    nfull = nslab * _FULL_CHUNKS

    def full_body(t, carry):
        chunk = wid + t * 32
        slab = chunk // _FULL_CHUNKS
        k = chunk - slab * _FULL_CHUNKS
        process_chunk(slab, k * _RCHUNK, _RCHUNK)
        return carry

    lax.fori_loop(0, nfull // 32, full_body, 0)

    def tail_body(t, carry):
        slab = wid + t * 32

        @pl.when(slab < nslab)
        def _():
            process_chunk(slab, _TAIL_R0, _G - _TAIL_R0)

        return carry

    lax.fori_loop(0, (nslab + 31) // 32, tail_body, 0)


def kernel(x, img_dim):
    B = x.shape[0]
    g = x.shape[2]
    s = g * g
    st = jnp.asarray(img_dim, jnp.float32) / g
    st16 = jnp.broadcast_to(st, (16,))

    nslab = B * 3
    mesh = plsc.VectorSubcoreMesh(core_axis_name="c", subcore_axis_name="s")
    sc = functools.partial(
        pl.kernel,
        mesh=mesh,
        compiler_params=pltpu.CompilerParams(needs_layout_passes=False),
        out_type=jax.ShapeDtypeStruct((B, 3 * s, _COUT), jnp.float32),
        scratch_types=[
            pltpu.VMEM((_NA, _RCHUNK, g), jnp.float32),
            pltpu.VMEM((_NB, _RCHUNK, g), jnp.float32),
            pltpu.VMEM((_HP, _COUT), jnp.float32),
            pltpu.VMEM((16,), jnp.float32),
            pltpu.SemaphoreType.DMA,
        ],
    )(functools.partial(_sc_body, nslab))
    out = sc(x, st16)
    return (out, 0)


# speedup vs baseline: 1.0913x; 1.0913x over previous
import functools

import jax
import jax.numpy as jnp
from jax import lax
from jax.experimental import pallas as pl
from jax.experimental.pallas import tpu as pltpu
from jax.experimental.pallas import tpu_sc as plsc

_ANCHOR_W = (116.0, 156.0, 373.0)
_ANCHOR_H = (90.0, 198.0, 326.0)
_CIN = 89
_COUT = 85
_G = 52
_RCHUNK = 8
_FULL_CHUNKS = _G // _RCHUNK
_TAIL_R0 = _FULL_CHUNKS * _RCHUNK
_NA = 56
_NB = _CIN - _NA


def _rcp(d):
    y = plsc.bitcast(jnp.int32(0x7EF311C3) - plsc.bitcast(d, jnp.int32),
                     jnp.float32)
    y = y * (2.0 - d * y)
    y = y * (2.0 - d * y)
    return y * (2.0 - d * y)


def _sigmoid(v):
    return _rcp(1.0 + jnp.exp(-v))


def _sc_body(nslab, x_hbm, st_hbm, out_hbm,
             bufa, bufb, ob0, ob1, stv, sem, so0, so1):
    nc = 2
    wid = lax.axis_index("s") * nc + lax.axis_index("c")
    pltpu.sync_copy(st_hbm, stv)
    i16 = lax.iota(jnp.int32, 16)

    def process_chunk(slab, r0, nrows):
        b = slab // 3
        a = slab - b * 3
        aw = jnp.where(a == 0, _ANCHOR_W[0],
                       jnp.where(a == 1, _ANCHOR_W[1], _ANCHOR_W[2]))
        ah = jnp.where(a == 0, _ANCHOR_H[0],
                       jnp.where(a == 1, _ANCHOR_H[1], _ANCHOR_H[2]))
        st = stv[:]
        n0 = a * (_G * _G) + r0 * _G

        dst_a = bufa if nrows == _RCHUNK else bufa.at[:, pl.ds(0, nrows)]
        dst_b = bufb if nrows == _RCHUNK else bufb.at[:, pl.ds(0, nrows)]
        ha = pltpu.async_copy(
            x_hbm.at[b, pl.ds(a * _CIN, _NA), pl.ds(r0, nrows), :], dst_a, sem)
        hb = pltpu.async_copy(
            x_hbm.at[b, pl.ds(a * _CIN + _NA, _NB), pl.ds(r0, nrows), :],
            dst_b, sem)
        ha.wait()
        hb.wait()

        def out_desc(q, ob, so):
            return pltpu.make_async_copy(
                ob, out_hbm.at[b, pl.ds(n0 + q * 16, 16), :], so)

        def block(q, ob):
            p = i16 + q * 16
            rv = p // _G
            lv = p - rv * _G
            z = jnp.full(16, 0, jnp.int32)

            s0 = _sigmoid(plsc.load_gather(bufa, [z, rv, lv]))
            s1 = _sigmoid(plsc.load_gather(bufa, [z + 1, rv, lv]))
            e2 = jnp.exp(plsc.load_gather(bufa, [z + 2, rv, lv]))
            e3 = jnp.exp(plsc.load_gather(bufa, [z + 3, rv, lv]))
            c84 = _sigmoid(plsc.load_gather(bufb, [z + 32, rv, lv]))
            plsc.store_scatter(ob, [i16, z],
                               (s0 + lv.astype(jnp.float32)) * st)
            plsc.store_scatter(ob, [i16, z + 1],
                               (s1 + (rv + r0).astype(jnp.float32)) * st)
            plsc.store_scatter(ob, [i16, z + 2], e2 * aw)
            plsc.store_scatter(ob, [i16, z + 3], e3 * ah)
            plsc.store_scatter(ob, [i16, z + 84], c84)

            @plsc.parallel_loop(0, _NA - 8, 1, unroll=4)
            def ca_loop(c):
                v = _sigmoid(plsc.load_gather(bufa, [z + (c + 8), rv, lv]))
                plsc.store_scatter(ob, [i16, z + (c + 4)], v)

            @plsc.parallel_loop(0, _NB - 1, 1, unroll=4)
            def cb_loop(c):
                v = _sigmoid(plsc.load_gather(bufb, [z + c, rv, lv]))
                plsc.store_scatter(ob, [i16, z + (c + 52)], v)

        nblk = (nrows * _G) // 16

        def q_body(q, carry):
            @pl.when(lax.rem(q, 2) == 0)
            def _():
                @pl.when(q >= 2)
                def _():
                    out_desc(q - 2, ob0, so0).wait()

                block(q, ob0)
                out_desc(q, ob0, so0).start()

            @pl.when(lax.rem(q, 2) == 1)
            def _():
                @pl.when(q >= 3)
                def _():
                    out_desc(q - 2, ob1, so1).wait()

                block(q, ob1)
                out_desc(q, ob1, so1).start()

            return carry

        lax.fori_loop(0, nblk, q_body, 0)
        out_desc(nblk - 2, ob0 if nblk % 2 == 0 else ob1,
                 so0 if nblk % 2 == 0 else so1).wait()
        out_desc(nblk - 1, ob1 if nblk % 2 == 0 else ob0,
                 so1 if nblk % 2 == 0 else so0).wait()

    nfull = nslab * _FULL_CHUNKS

    def full_body(t, carry):
        chunk = wid + t * 32
        slab = chunk // _FULL_CHUNKS
        k = chunk - slab * _FULL_CHUNKS
        process_chunk(slab, k * _RCHUNK, _RCHUNK)
        return carry

    lax.fori_loop(0, nfull // 32, full_body, 0)

    def tail_body(t, carry):
        slab = wid + t * 32

        @pl.when(slab < nslab)
        def _():
            process_chunk(slab, _TAIL_R0, _G - _TAIL_R0)

        return carry

    lax.fori_loop(0, (nslab + 31) // 32, tail_body, 0)


def kernel(x, img_dim):
    B = x.shape[0]
    g = x.shape[2]
    s = g * g
    st = jnp.asarray(img_dim, jnp.float32) / g
    st16 = jnp.broadcast_to(st, (16,))

    nslab = B * 3
    mesh = plsc.VectorSubcoreMesh(core_axis_name="c", subcore_axis_name="s")
    sc = functools.partial(
        pl.kernel,
        mesh=mesh,
        compiler_params=pltpu.CompilerParams(needs_layout_passes=False),
        out_type=jax.ShapeDtypeStruct((B, 3 * s, _COUT), jnp.float32),
        scratch_types=[
            pltpu.VMEM((_NA, _RCHUNK, g), jnp.float32),
            pltpu.VMEM((_NB, _RCHUNK, g), jnp.float32),
            pltpu.VMEM((16, _COUT), jnp.float32),
            pltpu.VMEM((16, _COUT), jnp.float32),
            pltpu.VMEM((16,), jnp.float32),
            pltpu.SemaphoreType.DMA,
            pltpu.SemaphoreType.DMA,
            pltpu.SemaphoreType.DMA,
        ],
    )(functools.partial(_sc_body, nslab))
    out = sc(x, st16)
    return (out, 0)
